# Initial kernel scaffold; baseline (speedup 1.0000x reference)
import jax
import jax.numpy as jnp
from jax.experimental import pallas as pl


def kernel(n_h, n_c, f_in, type_id, U_iou, b_iou, U_f, b_f):
    raise NotImplementedError("write your pallas kernel here")


# speedup vs baseline: 2.6986x; 2.6986x over previous
import functools

import jax
import jax.numpy as jnp
from jax.experimental import pallas as pl
from jax.experimental.pallas import tpu as pltpu

N = 10000
K = 32
H = 128
NT = 4
BLOCK_N = 200


def _tree_cell_kernel(oneh_ref, nh_ref, nc_ref, fin_ref,
                      uiou_ref, biou_ref, uf_ref, bf_ref,
                      iou_out, c_out):
    nh = nh_ref[...]
    oneh = oneh_ref[...]
    h_tilde = jnp.sum(nh, axis=1)
    nh2 = nh.reshape(BLOCK_N * K, H)

    iou = jnp.zeros((BLOCK_N, 3 * H), dtype=jnp.float32)
    fpre = jnp.zeros((BLOCK_N, K, H), dtype=jnp.float32)
    for t in range(NT):
        w = oneh[:, t]
        iou_t = jnp.dot(h_tilde, uiou_ref[t],
                        preferred_element_type=jnp.float32) + biou_ref[t:t + 1, :]
        iou = iou + w[:, None] * iou_t
        f_t = jnp.dot(nh2, uf_ref[t],
                      preferred_element_type=jnp.float32).reshape(BLOCK_N, K, H)
        f_t = f_t + bf_ref[t:t + 1, :][None, :, :]
        fpre = fpre + w[:, None, None] * f_t

    f = jax.nn.sigmoid(fpre + fin_ref[...][:, None, :])
    c_out[...] = jnp.sum(f * nc_ref[...], axis=1)
    iou_out[...] = iou


@jax.jit
def kernel(n_h, n_c, f_in, type_id, U_iou, b_iou, U_f, b_f):
    oneh = (type_id[:, None] == jnp.arange(NT, dtype=type_id.dtype)[None, :])
    oneh = oneh.astype(jnp.float32)

    grid = (N // BLOCK_N,)
    out = pl.pallas_call(
        _tree_cell_kernel,
        grid=grid,
        in_specs=[
            pl.BlockSpec((BLOCK_N, NT), lambda i: (i, 0)),
            pl.BlockSpec((BLOCK_N, K, H), lambda i: (i, 0, 0)),
            pl.BlockSpec((BLOCK_N, K, H), lambda i: (i, 0, 0)),
            pl.BlockSpec((BLOCK_N, H), lambda i: (i, 0)),
            pl.BlockSpec((NT, H, 3 * H), lambda i: (0, 0, 0)),
            pl.BlockSpec((NT, 3 * H), lambda i: (0, 0)),
            pl.BlockSpec((NT, H, H), lambda i: (0, 0, 0)),
            pl.BlockSpec((NT, H), lambda i: (0, 0)),
        ],
        out_specs=[
            pl.BlockSpec((BLOCK_N, 3 * H), lambda i: (i, 0)),
            pl.BlockSpec((BLOCK_N, H), lambda i: (i, 0)),
        ],
        out_shape=[
            jax.ShapeDtypeStruct((N, 3 * H), jnp.float32),
            jax.ShapeDtypeStruct((N, H), jnp.float32),
        ],
        compiler_params=pltpu.CompilerParams(
            dimension_semantics=("arbitrary",),
        ),
    )(oneh, n_h, n_c, f_in, U_iou, b_iou, U_f, b_f)
    return out[0], out[1]
